# Initial kernel scaffold; baseline (speedup 1.0000x reference)
import functools

import jax
import jax.numpy as jnp
from jax import lax
from jax.experimental import pallas as pl
from jax.experimental.pallas import tpu as pltpu
from jax.experimental.pallas import tpu_sc as plsc

N = 10000
E = 320000
D = 128
H = 8
Dh = 16
HDh = H * Dh
DOUT = 64
ALPHA = 0.2

NC = 2
NS = 16
NW = NC * NS
K = 80
CPW = E // (NW * K)
ROWS_PER_TILE = N // NS
ZR = 25


def _s1_body(x_ref, w1_ref, asrc_ref, adst_ref, h_ref, es_ref, ed_ref):
    hb = jnp.dot(x_ref[...], w1_ref[...], preferred_element_type=jnp.float32)
    h_ref[...] = hb
    es_ref[...] = jnp.dot(hb, asrc_ref[...], preferred_element_type=jnp.float32)
    ed_ref[...] = jnp.dot(hb, adst_ref[...], preferred_element_type=jnp.float32)


def _stage1(x, W1, Asrc, Adst):
    B = 400
    grid = (N // B,)
    return pl.pallas_call(
        _s1_body,
        grid=grid,
        in_specs=[
            pl.BlockSpec((B, D), lambda i: (i, 0)),
            pl.BlockSpec((D, HDh), lambda i: (0, 0)),
            pl.BlockSpec((HDh, 16), lambda i: (0, 0)),
            pl.BlockSpec((HDh, 16), lambda i: (0, 0)),
        ],
        out_specs=[
            pl.BlockSpec((B, HDh), lambda i: (i, 0)),
            pl.BlockSpec((B, 16), lambda i: (i, 0)),
            pl.BlockSpec((B, 16), lambda i: (i, 0)),
        ],
        out_shape=[
            jax.ShapeDtypeStruct((N, HDh), jnp.float32),
            jax.ShapeDtypeStruct((N, 16), jnp.float32),
            jax.ShapeDtypeStruct((N, 16), jnp.float32),
        ],
    )(x, W1, Asrc, Adst)


def _s2_body(src_hbm, dst_hbm, es_hbm, ed_hbm, h_hbm,
             agg_out, den_out,
             srcb, dstb, esb, edb, hb, zb, zb16,
             agg_sh, den_sh, sem1, sem2, sem3):
    c = lax.axis_index("c")
    s = lax.axis_index("s")
    wid = s * NC + c

    for r in range(ZR):
        for l in range(HDh // 16):
            zb[r, pl.ds(l * 16, 16)] = jnp.zeros((16,), jnp.float32)
        zb16[r, :] = jnp.zeros((16,), jnp.float32)
    rbase = s * ROWS_PER_TILE
    for i in range(ROWS_PER_TILE // ZR):
        pltpu.sync_copy(zb, agg_sh.at[pl.ds(rbase + i * ZR, ZR)])
        pltpu.sync_copy(zb16, den_sh.at[pl.ds(rbase + i * ZR, ZR)])
    plsc.subcore_barrier()

    def chunk_body(t, carry):
        base = (wid * CPW + t) * K
        pltpu.sync_copy(src_hbm.at[pl.ds(base, K)], srcb)
        pltpu.sync_copy(dst_hbm.at[pl.ds(base, K)], dstb)
        cp1 = pltpu.async_copy(es_hbm.at[srcb], esb, sem1)
        cp2 = pltpu.async_copy(ed_hbm.at[dstb], edb, sem2)
        cp3 = pltpu.async_copy(h_hbm.at[srcb], hb, sem3)
        cp1.wait()
        cp2.wait()
        cp3.wait()

        def edge_body(j, carry2):
            ev = esb[j, :] + edb[j, :]
            ev = jnp.where(ev >= 0.0, ev, ALPHA * ev)
            w = jnp.exp(ev)
            edb[j, :] = w
            for hh in range(H):
                ws = edb[j, hh]
                sl = pl.ds(hh * 16, 16)
                hb[j, sl] = hb[j, sl] * ws
            return carry2

        lax.fori_loop(0, K, edge_body, 0, unroll=2)

        pltpu.sync_copy(hb, agg_sh.at[dstb], add=True)
        pltpu.sync_copy(edb, den_sh.at[dstb], add=True)
        return carry

    lax.fori_loop(0, CPW, chunk_body, 0)
    plsc.subcore_barrier()

    pltpu.sync_copy(agg_sh.at[pl.ds(rbase, ROWS_PER_TILE)],
                    agg_out.at[c, pl.ds(rbase, ROWS_PER_TILE)])
    pltpu.sync_copy(den_sh.at[pl.ds(rbase, ROWS_PER_TILE)],
                    den_out.at[c, pl.ds(rbase, ROWS_PER_TILE)])


def _stage2(src, dst, es_t, ed_t, h):
    mesh = plsc.VectorSubcoreMesh(core_axis_name="c", subcore_axis_name="s")
    fn = pl.kernel(
        _s2_body,
        out_type=[
            jax.ShapeDtypeStruct((NC, N, HDh), jnp.float32),
            jax.ShapeDtypeStruct((NC, N, 16), jnp.float32),
        ],
        mesh=mesh,
        scratch_types=[
            pltpu.VMEM((K,), jnp.int32),
            pltpu.VMEM((K,), jnp.int32),
            pltpu.VMEM((K, 16), jnp.float32),
            pltpu.VMEM((K, 16), jnp.float32),
            pltpu.VMEM((K, HDh), jnp.float32),
            pltpu.VMEM((ZR, HDh), jnp.float32),
            pltpu.VMEM((ZR, 16), jnp.float32),
            pltpu.VMEM_SHARED((N, HDh), jnp.float32),
            pltpu.VMEM_SHARED((N, 16), jnp.float32),
            pltpu.SemaphoreType.DMA,
            pltpu.SemaphoreType.DMA,
            pltpu.SemaphoreType.DMA,
        ],
    )
    return fn(src, dst, es_t, ed_t, h)


def _s3_body(agg_ref, den_ref, wout_ref, bout_ref, r_ref, emb_ref):
    agg = agg_ref[0] + agg_ref[1]
    den = den_ref[0] + den_ref[1]
    deninv = 1.0 / (den[:, 0:H] + 1e-16)
    den128 = jnp.dot(deninv, r_ref[...],
                     preferred_element_type=jnp.float32)
    out = agg * den128
    out = jnp.where(out > 0.0, out, jnp.expm1(out))
    emb = jnp.dot(out, wout_ref[...],
                  preferred_element_type=jnp.float32) + bout_ref[...]
    nrm = jnp.sqrt(jnp.sum(emb * emb, axis=1, keepdims=True))
    emb_ref[...] = emb / (nrm + 1e-12)


def _stage3(aggp, denp, W_out, b_out, R):
    B = 400
    grid = (N // B,)
    return pl.pallas_call(
        _s3_body,
        grid=grid,
        in_specs=[
            pl.BlockSpec((NC, B, HDh), lambda i: (0, i, 0)),
            pl.BlockSpec((NC, B, 16), lambda i: (0, i, 0)),
            pl.BlockSpec((HDh, DOUT), lambda i: (0, 0)),
            pl.BlockSpec((1, DOUT), lambda i: (0, 0)),
            pl.BlockSpec((H, HDh), lambda i: (0, 0)),
        ],
        out_specs=pl.BlockSpec((B, DOUT), lambda i: (i, 0)),
        out_shape=jax.ShapeDtypeStruct((N, DOUT), jnp.float32),
    )(aggp, denp, W_out, b_out, R)


@jax.jit
def kernel(x, edge_index, W1, a_src, a_dst, W_out, b_out):
    src = edge_index[0]
    dst = edge_index[1]
    rows = jnp.arange(HDh)
    Asrc = jnp.zeros((HDh, 16), jnp.float32).at[rows, rows // Dh].set(
        a_src.reshape(HDh))
    Adst = jnp.zeros((HDh, 16), jnp.float32).at[rows, rows // Dh].set(
        a_dst.reshape(HDh))
    R = jnp.kron(jnp.eye(H, dtype=jnp.float32),
                 jnp.ones((1, Dh), jnp.float32))
    h, es_t, ed_t = _stage1(x, W1, Asrc, Adst)
    aggp, denp = _stage2(src, dst, es_t, ed_t, h)
    return _stage3(aggp, denp, W_out, b_out, R)


# speedup vs baseline: 47.1169x; 47.1169x over previous
import functools

import jax
import jax.numpy as jnp
from jax import lax
from jax.experimental import pallas as pl
from jax.experimental.pallas import tpu as pltpu
from jax.experimental.pallas import tpu_sc as plsc

N = 10000
E = 320000
D = 128
H = 8
Dh = 16
HDh = H * Dh
DOUT = 64
ALPHA = 0.2

NC = 2
NS = 16
NW = NC * NS
K = 80
CPW = E // (NW * K)
NPAD = 10240
ROWS_PER_TILE = NPAD // NS
ZR = 32


def _s1_body(x_ref, w1_ref, acomb_ref, h_ref, esed_ref):
    hb = jnp.dot(x_ref[...], w1_ref[...], preferred_element_type=jnp.float32)
    h_ref[...] = hb
    esed_ref[...] = jnp.dot(hb, acomb_ref[...], preferred_element_type=jnp.float32)


def _stage1(x, W1, Acomb):
    B = 400
    grid = (N // B,)
    return pl.pallas_call(
        _s1_body,
        grid=grid,
        in_specs=[
            pl.BlockSpec((B, D), lambda i: (i, 0)),
            pl.BlockSpec((D, HDh), lambda i: (0, 0)),
            pl.BlockSpec((HDh, 16), lambda i: (0, 0)),
        ],
        out_specs=[
            pl.BlockSpec((B, HDh), lambda i: (i, 0)),
            pl.BlockSpec((B, 16), lambda i: (i, 0)),
        ],
        out_shape=[
            jax.ShapeDtypeStruct((N, HDh), jnp.float32),
            jax.ShapeDtypeStruct((N, 16), jnp.float32),
        ],
    )(x, W1, Acomb)


def _vgather(v, idx):
    return lax.gather(
        v, idx[:, None],
        lax.GatherDimensionNumbers(offset_dims=(), collapsed_slice_dims=(0,),
                                   start_index_map=(0,)),
        slice_sizes=(1,),
        mode=lax.GatherScatterMode.PROMISE_IN_BOUNDS)


def _s2_body(src_hbm, dst_hbm, esed_hbm, h_hbm,
             agg_out, den_out,
             srcb, dstb, cidxb, didxb, egb, wb, hb, zb2, zbf,
             agg_sh, den_sh, sem1, sem2):
    c = lax.axis_index("c")
    s = lax.axis_index("s")
    wid = s * NC + c
    iota = lax.iota(jnp.int32, 16)
    lo8 = iota < 8

    for r in range(8):
        for l in range(HDh // 16):
            zb2[r, pl.ds(l * 16, 16)] = jnp.zeros((16,), jnp.float32)
    for r in range(64):
        zbf[pl.ds(r * 16, 16)] = jnp.zeros((16,), jnp.float32)
    rbase = s * ROWS_PER_TILE
    for i in range(ROWS_PER_TILE // 8):
        pltpu.sync_copy(zb2, agg_sh.at[pl.ds(rbase + i * 8, 8)])
    fbase = s * (NPAD * 16 // NS)
    for i in range(NPAD * 16 // NS // 1024):
        pltpu.sync_copy(zbf, den_sh.at[pl.ds(fbase + i * 1024, 1024)])
    plsc.subcore_barrier()

    def chunk_body(t, carry):
        base = (wid * CPW + t) * K
        pltpu.sync_copy(src_hbm.at[pl.ds(base, K)], srcb)
        pltpu.sync_copy(dst_hbm.at[pl.ds(base, K)], dstb)

        def build_body(m, carry2):
            sv = srcb[pl.ds(m * 16, 16)]
            dv = dstb[pl.ds(m * 16, 16)]
            for l in range(16):
                lane = jnp.full((16,), l, jnp.int32)
                svl = _vgather(sv, lane)
                dvl = _vgather(dv, lane)
                sel = jnp.where(lo8, svl, dvl)
                off = (m * 16 + l) * 16
                cidxb[pl.ds(off, 16)] = sel * 16 + iota
                didxb[pl.ds(off, 16)] = dvl * 16 + iota
            return carry2

        lax.fori_loop(0, K // 16, build_body, 0)

        cp1 = pltpu.async_copy(esed_hbm.at[cidxb], egb, sem1)
        cp2 = pltpu.async_copy(h_hbm.at[srcb], hb, sem2)
        cp1.wait()
        cp2.wait()

        swap8 = iota ^ 8
        def edge_body(m, carry2):
            for l in range(16):
                j = m * 16 + l
                eg = egb[pl.ds(j * 16, 16)]
                ev = eg + _vgather(eg, swap8)
                ev = jnp.where(ev >= 0.0, ev, ALPHA * ev)
                w = jnp.exp(ev)
                wb[pl.ds(j * 16, 16)] = w
                for hh in range(H):
                    wspl = _vgather(w, jnp.full((16,), hh, jnp.int32))
                    sl = pl.ds(hh * 16, 16)
                    hb[j, sl] = hb[j, sl] * wspl
            return carry2

        lax.fori_loop(0, K // 16, edge_body, 0)

        pltpu.sync_copy(hb, agg_sh.at[dstb], add=True)
        pltpu.sync_copy(wb, den_sh.at[didxb], add=True)
        return carry

    lax.fori_loop(0, CPW, chunk_body, 0)
    plsc.subcore_barrier()

    pltpu.sync_copy(agg_sh.at[pl.ds(rbase, ROWS_PER_TILE)],
                    agg_out.at[c, pl.ds(rbase, ROWS_PER_TILE)])
    pltpu.sync_copy(den_sh.at[pl.ds(fbase, NPAD * 16 // NS)],
                    den_out.at[c, pl.ds(fbase, NPAD * 16 // NS)])


def _stage2(src, dst, esed_flat, h):
    mesh = plsc.VectorSubcoreMesh(core_axis_name="c", subcore_axis_name="s")
    fn = pl.kernel(
        _s2_body,
        out_type=[
            jax.ShapeDtypeStruct((NC, NPAD, HDh), jnp.float32),
            jax.ShapeDtypeStruct((NC, NPAD * 16), jnp.float32),
        ],
        mesh=mesh,
        scratch_types=[
            pltpu.VMEM((K,), jnp.int32),
            pltpu.VMEM((K,), jnp.int32),
            pltpu.VMEM((K * 16,), jnp.int32),
            pltpu.VMEM((K * 16,), jnp.int32),
            pltpu.VMEM((K * 16,), jnp.float32),
            pltpu.VMEM((K * 16,), jnp.float32),
            pltpu.VMEM((K, HDh), jnp.float32),
            pltpu.VMEM((8, HDh), jnp.float32),
            pltpu.VMEM((1024,), jnp.float32),
            pltpu.VMEM_SHARED((NPAD, HDh), jnp.float32),
            pltpu.VMEM_SHARED((NPAD * 16,), jnp.float32),
            pltpu.SemaphoreType.DMA,
            pltpu.SemaphoreType.DMA,
        ],
    )
    return fn(src, dst, esed_flat, h)


def _s3_body(agg_ref, den_ref, wout_ref, bout_ref, r_ref, emb_ref):
    agg = agg_ref[0] + agg_ref[1]
    den = den_ref[0] + den_ref[1]
    deninv = 1.0 / (den[:, 0:H] + 1e-16)
    den128 = jnp.dot(deninv, r_ref[...],
                     preferred_element_type=jnp.float32)
    out = agg * den128
    out = jnp.where(out > 0.0, out, jnp.exp(jnp.minimum(out, 0.0)) - 1.0)

    emb = jnp.dot(out, wout_ref[...],
                  preferred_element_type=jnp.float32) + bout_ref[...]
    nrm = jnp.sqrt(jnp.sum(emb * emb, axis=1, keepdims=True))
    emb_ref[...] = emb / (nrm + 1e-12)


def _stage3(aggp, denp, W_out, b_out, R):
    B = 400
    grid = (N // B,)
    return pl.pallas_call(
        _s3_body,
        grid=grid,
        in_specs=[
            pl.BlockSpec((NC, B, HDh), lambda i: (0, i, 0)),
            pl.BlockSpec((NC, B, 16), lambda i: (0, i, 0)),
            pl.BlockSpec((HDh, DOUT), lambda i: (0, 0)),
            pl.BlockSpec((1, DOUT), lambda i: (0, 0)),
            pl.BlockSpec((H, HDh), lambda i: (0, 0)),
        ],
        out_specs=pl.BlockSpec((B, DOUT), lambda i: (i, 0)),
        out_shape=jax.ShapeDtypeStruct((N, DOUT), jnp.float32),
    )(aggp, denp, W_out, b_out.reshape(1, DOUT), R)


@jax.jit
def kernel(x, edge_index, W1, a_src, a_dst, W_out, b_out):
    src = edge_index[0]
    dst = edge_index[1]
    rows = jnp.arange(HDh)
    Acomb = (jnp.zeros((HDh, 16), jnp.float32)
             .at[rows, rows // Dh].set(a_src.reshape(HDh))
             .at[rows, 8 + rows // Dh].set(a_dst.reshape(HDh)))
    R = jnp.kron(jnp.eye(H, dtype=jnp.float32),
                 jnp.ones((1, Dh), jnp.float32))
    h, esed = _stage1(x, W1, Acomb)
    aggp, denf = _stage2(src, dst, esed.reshape(N * 16), h)
    denp = denf.reshape(NC, NPAD, 16)
    return _stage3(aggp, denp, W_out, b_out.reshape(1, DOUT), R)


# speedup vs baseline: 79.4357x; 1.6859x over previous
import functools

import jax
import jax.numpy as jnp
from jax import lax
from jax.experimental import pallas as pl
from jax.experimental.pallas import tpu as pltpu
from jax.experimental.pallas import tpu_sc as plsc

N = 10000
E = 320000
D = 128
H = 8
Dh = 16
HDh = H * Dh
DOUT = 64
ALPHA = 0.2

NC = 2
NS = 16
NW = NC * NS
K = 80
CPW = E // (NW * K)
NPAD = 10112
KW = K * 16
NBUF = 3
ROWS_PER_TILE = NPAD // NS
ZR = 32


def _s1_body(x_ref, w1_ref, acomb_ref, h_ref, esed_ref):
    hb = jnp.dot(x_ref[...], w1_ref[...], preferred_element_type=jnp.float32)
    h_ref[...] = hb
    esed_ref[...] = jnp.dot(hb, acomb_ref[...], preferred_element_type=jnp.float32)


def _stage1(x, W1, Acomb):
    B = 400
    grid = (N // B,)
    return pl.pallas_call(
        _s1_body,
        grid=grid,
        in_specs=[
            pl.BlockSpec((B, D), lambda i: (i, 0)),
            pl.BlockSpec((D, HDh), lambda i: (0, 0)),
            pl.BlockSpec((HDh, 16), lambda i: (0, 0)),
        ],
        out_specs=[
            pl.BlockSpec((B, HDh), lambda i: (i, 0)),
            pl.BlockSpec((B, 16), lambda i: (i, 0)),
        ],
        out_shape=[
            jax.ShapeDtypeStruct((N, HDh), jnp.float32),
            jax.ShapeDtypeStruct((N, 16), jnp.float32),
        ],
    )(x, W1, Acomb)


def _vgather(v, idx):
    return lax.gather(
        v, idx[:, None],
        lax.GatherDimensionNumbers(offset_dims=(), collapsed_slice_dims=(0,),
                                   start_index_map=(0,)),
        slice_sizes=(1,),
        mode=lax.GatherScatterMode.PROMISE_IN_BOUNDS)


def _s2_body(src_hbm, dst_hbm, esed_hbm, h_hbm, z2d_hbm, z1d_hbm,
             agg_out, den_out,
             srcb0, srcb1, srcb2, dstb0, dstb1, dstb2,
             cidxb0, cidxb1, cidxb2,
             egb0, egb1, egb2, hb0, hb1, hb2,
             agg_sh, den_sh,
             gsem0, gsem1, gsem2, ssem0, ssem1, ssem2):
    srcb = [srcb0, srcb1, srcb2]
    dstb = [dstb0, dstb1, dstb2]
    cidxb = [cidxb0, cidxb1, cidxb2]
    egb = [egb0, egb1, egb2]
    hb = [hb0, hb1, hb2]
    gsems = [gsem0, gsem1, gsem2]
    ssems = [ssem0, ssem1, ssem2]
    c = lax.axis_index("c")
    s = lax.axis_index("s")
    wid = s * NC + c
    iota = lax.iota(jnp.int32, 16)
    lo8 = iota < 8
    swap8 = iota ^ 8

    rbase = s * ROWS_PER_TILE
    fbase = s * (NPAD * 16 // NS)
    pltpu.sync_copy(z2d_hbm, agg_sh.at[pl.ds(rbase, ROWS_PER_TILE)])
    pltpu.sync_copy(z1d_hbm, den_sh.at[pl.ds(fbase, NPAD * 16 // NS)])
    plsc.subcore_barrier()

    def prep(t, b):
        base = (wid * CPW + t) * K
        pltpu.sync_copy(src_hbm.at[pl.ds(base, K)], srcb[b])
        pltpu.sync_copy(dst_hbm.at[pl.ds(base, K)], dstb[b])

        def build_body(m, carry2):
            sv = srcb[b][pl.ds(m * 16, 16)]
            dv = dstb[b][pl.ds(m * 16, 16)]
            for l in range(16):
                lane = jnp.full((16,), l, jnp.int32)
                svl = _vgather(sv, lane)
                dvl = _vgather(dv, lane)
                sel = jnp.where(lo8, svl, dvl)
                off = (m * 16 + l) * 16
                cidxb[b][pl.ds(off, 16)] = sel * 16 + iota
            return carry2

        lax.fori_loop(0, K // 16, build_body, 0)
        pltpu.async_copy(esed_hbm.at[cidxb[b]], egb[b], gsems[b])
        pltpu.async_copy(h_hbm.at[srcb[b]], hb[b], gsems[b])

    def wait_gathers(b):
        pltpu.make_async_copy(esed_hbm.at[cidxb[b]], egb[b], gsems[b]).wait()
        pltpu.make_async_copy(h_hbm.at[srcb[b]], hb[b], gsems[b]).wait()

    def compute(b):
        def edge_body(m, carry2):
            for l in range(16):
                j = m * 16 + l
                eg = egb[b][pl.ds(j * 16, 16)]
                ev = eg + _vgather(eg, swap8)
                ev = jnp.where(ev >= 0.0, ev, ALPHA * ev)
                w = jnp.exp(ev)
                egb[b][pl.ds(j * 16, 16)] = w
                for hh in range(H):
                    wspl = _vgather(w, jnp.full((16,), hh, jnp.int32))
                    sl = pl.ds(hh * 16, 16)
                    hb[b][j, sl] = hb[b][j, sl] * wspl
            return carry2

        lax.fori_loop(0, K // 16, edge_body, 0)

    def start_scatters(b):
        pltpu.async_copy(hb[b], agg_sh.at[dstb[b]], ssems[b], add=True)
        pltpu.async_copy(egb[b], den_sh.at[cidxb[b]], ssems[b], add=True)

    def wait_scatters(b):
        pltpu.make_async_copy(hb[b], agg_sh.at[dstb[b]], ssems[b]).wait()
        pltpu.make_async_copy(egb[b], den_sh.at[cidxb[b]], ssems[b]).wait()

    prep(0, 0)

    def loop_body(p, carry):
        for q in range(3):
            t = 3 * p + 1 + q
            r = (1 + q) % 3
            rp = q % 3
            if q < 2:
                @pl.when(p > 0)
                def _():
                    wait_scatters(r)
            else:
                wait_scatters(r)
            prep(t, r)
            wait_gathers(rp)
            compute(rp)
            start_scatters(rp)
        return carry

    lax.fori_loop(0, (CPW - 2) // 3, loop_body, 0)

    wait_scatters(1)
    prep(CPW - 1, 1)
    wait_gathers(0)
    compute(0)
    start_scatters(0)

    wait_gathers(1)
    compute(1)
    start_scatters(1)
    wait_scatters(2)
    wait_scatters(0)
    wait_scatters(1)
    plsc.subcore_barrier()

    pltpu.sync_copy(agg_sh.at[pl.ds(rbase, ROWS_PER_TILE)],
                    agg_out.at[c, pl.ds(rbase, ROWS_PER_TILE)])
    pltpu.sync_copy(den_sh.at[pl.ds(fbase, NPAD * 16 // NS)],
                    den_out.at[c, pl.ds(fbase, NPAD * 16 // NS)])


def _stage2(src, dst, esed_flat, h):
    z2d = jnp.zeros((ROWS_PER_TILE, HDh), jnp.float32)
    z1d = jnp.zeros((NPAD * 16 // NS,), jnp.float32)
    mesh = plsc.VectorSubcoreMesh(core_axis_name="c", subcore_axis_name="s")
    fn = pl.kernel(
        _s2_body,
        out_type=[
            jax.ShapeDtypeStruct((NC, NPAD, HDh), jnp.float32),
            jax.ShapeDtypeStruct((NC, NPAD * 16), jnp.float32),
        ],
        mesh=mesh,
        scratch_types=(
            [pltpu.VMEM((K,), jnp.int32)] * 6
            + [pltpu.VMEM((KW,), jnp.int32)] * 3
            + [pltpu.VMEM((KW,), jnp.float32)] * 3
            + [pltpu.VMEM((K, HDh), jnp.float32)] * 3
            + [pltpu.VMEM_SHARED((NPAD, HDh), jnp.float32),
               pltpu.VMEM_SHARED((NPAD * 16,), jnp.float32)]
            + [pltpu.SemaphoreType.DMA] * 6
        ),
    )
    return fn(src, dst, esed_flat, h, z2d, z1d)


def _s3_body(agg_ref, den_ref, wout_ref, bout_ref, r_ref, emb_ref):
    agg = agg_ref[0] + agg_ref[1]
    den = den_ref[0] + den_ref[1]
    deninv = 1.0 / (den[:, H:16] + 1e-16)
    den128 = jnp.dot(deninv, r_ref[...],
                     preferred_element_type=jnp.float32)
    out = agg * den128
    out = jnp.where(out > 0.0, out, jnp.exp(jnp.minimum(out, 0.0)) - 1.0)

    emb = jnp.dot(out, wout_ref[...],
                  preferred_element_type=jnp.float32) + bout_ref[...]
    nrm = jnp.sqrt(jnp.sum(emb * emb, axis=1, keepdims=True))
    emb_ref[...] = emb / (nrm + 1e-12)


def _stage3(aggp, denp, W_out, b_out, R):
    B = 400
    grid = (N // B,)
    return pl.pallas_call(
        _s3_body,
        grid=grid,
        in_specs=[
            pl.BlockSpec((NC, B, HDh), lambda i: (0, i, 0)),
            pl.BlockSpec((NC, B, 16), lambda i: (0, i, 0)),
            pl.BlockSpec((HDh, DOUT), lambda i: (0, 0)),
            pl.BlockSpec((1, DOUT), lambda i: (0, 0)),
            pl.BlockSpec((H, HDh), lambda i: (0, 0)),
        ],
        out_specs=pl.BlockSpec((B, DOUT), lambda i: (i, 0)),
        out_shape=jax.ShapeDtypeStruct((N, DOUT), jnp.float32),
    )(aggp, denp, W_out, b_out.reshape(1, DOUT), R)


@jax.jit
def kernel(x, edge_index, W1, a_src, a_dst, W_out, b_out):
    src = edge_index[0]
    dst = edge_index[1]
    rows = jnp.arange(HDh)
    Acomb = (jnp.zeros((HDh, 16), jnp.float32)
             .at[rows, rows // Dh].set(a_src.reshape(HDh))
             .at[rows, 8 + rows // Dh].set(a_dst.reshape(HDh)))
    R = jnp.kron(jnp.eye(H, dtype=jnp.float32),
                 jnp.ones((1, Dh), jnp.float32))
    h, esed = _stage1(x, W1, Acomb)
    aggp, denf = _stage2(src, dst, esed.reshape(N * 16), h)
    denp = denf.reshape(NC, NPAD, 16)
    return _stage3(aggp, denp, W_out, b_out.reshape(1, DOUT), R)


# speedup vs baseline: 92.9330x; 1.1699x over previous
import functools

import jax
import jax.numpy as jnp
from jax import lax
from jax.experimental import pallas as pl
from jax.experimental.pallas import tpu as pltpu
from jax.experimental.pallas import tpu_sc as plsc

N = 10000
E = 320000
D = 128
H = 8
Dh = 16
HDh = H * Dh
DOUT = 64
ALPHA = 0.2

NC = 2
NS = 16
NW = NC * NS
K = 80
CPW = E // (NW * K)
NPAD = 10112
KW = K * 16
NBUF = 3
ROWS_PER_TILE = NPAD // NS
ZR = 32


def _s1_body(x_ref, w1_ref, acomb_ref, h_ref, esed_ref):
    hb = jnp.dot(x_ref[...], w1_ref[...], preferred_element_type=jnp.float32)
    h_ref[...] = hb
    esed_ref[...] = jnp.dot(hb, acomb_ref[...], preferred_element_type=jnp.float32)


def _stage1(x, W1, Acomb):
    B = 400
    grid = (N // B,)
    return pl.pallas_call(
        _s1_body,
        grid=grid,
        in_specs=[
            pl.BlockSpec((B, D), lambda i: (i, 0)),
            pl.BlockSpec((D, HDh), lambda i: (0, 0)),
            pl.BlockSpec((HDh, 16), lambda i: (0, 0)),
        ],
        out_specs=[
            pl.BlockSpec((B, HDh), lambda i: (i, 0)),
            pl.BlockSpec((B, 16), lambda i: (i, 0)),
        ],
        out_shape=[
            jax.ShapeDtypeStruct((N, HDh), jnp.float32),
            jax.ShapeDtypeStruct((N, 16), jnp.float32),
        ],
    )(x, W1, Acomb)


def _vgather(v, idx):
    return lax.gather(
        v, idx[:, None],
        lax.GatherDimensionNumbers(offset_dims=(), collapsed_slice_dims=(0,),
                                   start_index_map=(0,)),
        slice_sizes=(1,),
        mode=lax.GatherScatterMode.PROMISE_IN_BOUNDS)


def _s2_body(src_hbm, dst_hbm, esed_hbm, h_hbm, z2d_hbm, z1d_hbm,
             agg_out, den_out,
             srcb0, srcb1, srcb2, dstb0, dstb1, dstb2,
             cidxb0, cidxb1, cidxb2, didxb0, didxb1, didxb2,
             esg0, esg1, esg2, edg0, edg1, edg2, hb0, hb1, hb2,
             agg_sh, den_sh,
             gsem0, gsem1, gsem2, ssem0, ssem1, ssem2):
    srcb = [srcb0, srcb1, srcb2]
    dstb = [dstb0, dstb1, dstb2]
    cidxb = [cidxb0, cidxb1, cidxb2]
    didxb = [didxb0, didxb1, didxb2]
    esg = [esg0, esg1, esg2]
    edg = [edg0, edg1, edg2]
    hb = [hb0, hb1, hb2]
    gsems = [gsem0, gsem1, gsem2]
    ssems = [ssem0, ssem1, ssem2]
    c = lax.axis_index("c")
    s = lax.axis_index("s")
    wid = s * NC + c
    iota = lax.iota(jnp.int32, 16)
    io7 = iota & 7
    io78 = io7 + 8

    rbase = s * ROWS_PER_TILE
    fbase = s * (NPAD * 16 // NS)
    pltpu.sync_copy(z2d_hbm, agg_sh.at[pl.ds(rbase, ROWS_PER_TILE)])
    pltpu.sync_copy(z1d_hbm, den_sh.at[pl.ds(fbase, NPAD * 16 // NS)])
    plsc.subcore_barrier()

    def prep(t, b):
        base = (wid * CPW + t) * K
        pltpu.sync_copy(src_hbm.at[pl.ds(base, K)], srcb[b])
        pltpu.sync_copy(dst_hbm.at[pl.ds(base, K)], dstb[b])

        def build_body(m, carry2):
            sv = srcb[b][pl.ds(m * 16, 16)]
            dv = dstb[b][pl.ds(m * 16, 16)]
            for p8 in range(8):
                pc = jnp.where(iota < 8,
                               jnp.full((16,), 2 * p8, jnp.int32),
                               jnp.full((16,), 2 * p8 + 1, jnp.int32))
                srep = _vgather(sv, pc)
                drep = _vgather(dv, pc)
                off = (m * 8 + p8) * 16
                cidxb[b][pl.ds(off, 16)] = srep * 16 + io7
                didxb[b][pl.ds(off, 16)] = drep * 16 + io78
            return carry2

        lax.fori_loop(0, K // 16, build_body, 0)
        pltpu.async_copy(esed_hbm.at[cidxb[b]], esg[b], gsems[b])
        pltpu.async_copy(esed_hbm.at[didxb[b]], edg[b], gsems[b])
        pltpu.async_copy(h_hbm.at[srcb[b]], hb[b], gsems[b])

    def wait_gathers(b):
        pltpu.make_async_copy(esed_hbm.at[cidxb[b]], esg[b], gsems[b]).wait()
        pltpu.make_async_copy(esed_hbm.at[didxb[b]], edg[b], gsems[b]).wait()
        pltpu.make_async_copy(h_hbm.at[srcb[b]], hb[b], gsems[b]).wait()

    def compute(b):
        def edge_body(m, carry2):
            for p8 in range(8):
                off = (m * 8 + p8) * 16
                ev = esg[b][pl.ds(off, 16)] + edg[b][pl.ds(off, 16)]
                ev = jnp.where(ev >= 0.0, ev, ALPHA * ev)
                w = jnp.exp(ev)
                edg[b][pl.ds(off, 16)] = w
                ja = m * 16 + 2 * p8
                for hh in range(H):
                    sl = pl.ds(hh * 16, 16)
                    wsa = _vgather(w, jnp.full((16,), hh, jnp.int32))
                    hb[b][ja, sl] = hb[b][ja, sl] * wsa
                    wsb = _vgather(w, jnp.full((16,), 8 + hh, jnp.int32))
                    hb[b][ja + 1, sl] = hb[b][ja + 1, sl] * wsb
            return carry2

        lax.fori_loop(0, K // 16, edge_body, 0)

    def start_scatters(b):
        pltpu.async_copy(hb[b], agg_sh.at[dstb[b]], ssems[b], add=True)
        pltpu.async_copy(edg[b], den_sh.at[didxb[b]], ssems[b], add=True)

    def wait_scatters(b):
        pltpu.make_async_copy(hb[b], agg_sh.at[dstb[b]], ssems[b]).wait()
        pltpu.make_async_copy(edg[b], den_sh.at[didxb[b]], ssems[b]).wait()

    prep(0, 0)

    def loop_body(p, carry):
        for q in range(3):
            t = 3 * p + 1 + q
            r = (1 + q) % 3
            rp = q % 3
            if q < 2:
                @pl.when(p > 0)
                def _():
                    wait_scatters(r)
            else:
                wait_scatters(r)
            prep(t, r)
            wait_gathers(rp)
            compute(rp)
            start_scatters(rp)
        return carry

    lax.fori_loop(0, (CPW - 2) // 3, loop_body, 0)

    wait_scatters(1)
    prep(CPW - 1, 1)
    wait_gathers(0)
    compute(0)
    start_scatters(0)

    wait_gathers(1)
    compute(1)
    start_scatters(1)
    wait_scatters(2)
    wait_scatters(0)
    wait_scatters(1)
    plsc.subcore_barrier()

    pltpu.sync_copy(agg_sh.at[pl.ds(rbase, ROWS_PER_TILE)],
                    agg_out.at[c, pl.ds(rbase, ROWS_PER_TILE)])
    pltpu.sync_copy(den_sh.at[pl.ds(fbase, NPAD * 16 // NS)],
                    den_out.at[c, pl.ds(fbase, NPAD * 16 // NS)])


def _stage2(src, dst, esed_flat, h):
    z2d = jnp.zeros((ROWS_PER_TILE, HDh), jnp.float32)
    z1d = jnp.zeros((NPAD * 16 // NS,), jnp.float32)
    mesh = plsc.VectorSubcoreMesh(core_axis_name="c", subcore_axis_name="s")
    fn = pl.kernel(
        _s2_body,
        out_type=[
            jax.ShapeDtypeStruct((NC, NPAD, HDh), jnp.float32),
            jax.ShapeDtypeStruct((NC, NPAD * 16), jnp.float32),
        ],
        mesh=mesh,
        scratch_types=(
            [pltpu.VMEM((K,), jnp.int32)] * 6
            + [pltpu.VMEM((K * 8,), jnp.int32)] * 6
            + [pltpu.VMEM((K * 8,), jnp.float32)] * 6
            + [pltpu.VMEM((K, HDh), jnp.float32)] * 3
            + [pltpu.VMEM_SHARED((NPAD, HDh), jnp.float32),
               pltpu.VMEM_SHARED((NPAD * 16,), jnp.float32)]
            + [pltpu.SemaphoreType.DMA] * 6
        ),
    )
    return fn(src, dst, esed_flat, h, z2d, z1d)


def _s3_body(agg_ref, den_ref, wout_ref, bout_ref, r_ref, emb_ref):
    agg = agg_ref[0] + agg_ref[1]
    den = den_ref[0] + den_ref[1]
    deninv = 1.0 / (den[:, H:16] + 1e-16)
    den128 = jnp.dot(deninv, r_ref[...],
                     preferred_element_type=jnp.float32)
    out = agg * den128
    out = jnp.where(out > 0.0, out, jnp.exp(jnp.minimum(out, 0.0)) - 1.0)

    emb = jnp.dot(out, wout_ref[...],
                  preferred_element_type=jnp.float32) + bout_ref[...]
    nrm = jnp.sqrt(jnp.sum(emb * emb, axis=1, keepdims=True))
    emb_ref[...] = emb / (nrm + 1e-12)


def _stage3(aggp, denp, W_out, b_out, R):
    B = 400
    grid = (N // B,)
    return pl.pallas_call(
        _s3_body,
        grid=grid,
        in_specs=[
            pl.BlockSpec((NC, B, HDh), lambda i: (0, i, 0)),
            pl.BlockSpec((NC, B, 16), lambda i: (0, i, 0)),
            pl.BlockSpec((HDh, DOUT), lambda i: (0, 0)),
            pl.BlockSpec((1, DOUT), lambda i: (0, 0)),
            pl.BlockSpec((H, HDh), lambda i: (0, 0)),
        ],
        out_specs=pl.BlockSpec((B, DOUT), lambda i: (i, 0)),
        out_shape=jax.ShapeDtypeStruct((N, DOUT), jnp.float32),
    )(aggp, denp, W_out, b_out.reshape(1, DOUT), R)


@jax.jit
def kernel(x, edge_index, W1, a_src, a_dst, W_out, b_out):
    src = edge_index[0]
    dst = edge_index[1]
    rows = jnp.arange(HDh)
    Acomb = (jnp.zeros((HDh, 16), jnp.float32)
             .at[rows, rows // Dh].set(a_src.reshape(HDh))
             .at[rows, 8 + rows // Dh].set(a_dst.reshape(HDh)))
    R = jnp.kron(jnp.eye(H, dtype=jnp.float32),
                 jnp.ones((1, Dh), jnp.float32))
    h, esed = _stage1(x, W1, Acomb)
    aggp, denf = _stage2(src, dst, esed.reshape(N * 16), h)
    denp = denf.reshape(NC, NPAD, 16)
    return _stage3(aggp, denp, W_out, b_out.reshape(1, DOUT), R)
